# baseline (speedup 1.0000x reference)
import jax
import jax.numpy as jnp
from jax.experimental import pallas as pl
from jax.experimental.pallas import tpu as pltpu

NA = 5
NC = 20
NGT = 20
GY = 52
GX = 52
NPOS = GY * GX
CH = 25
IOU_TH = 0.6
L_OBJ = 5.0
L_PRIOR = 0.01
EPS = 1e-05
BIG = 1e9


def _mm(lhs_bf, rhs_bf):
    return jax.lax.dot_general(lhs_bf, rhs_bf, (((1,), (0,)), ((), ())),
                               preferred_element_type=jnp.float32)


def _body(pred_ref, gt_ref, gtt_ref, anchv_ref, cxy_ref, anch_ref, seen_ref,
          out_ref, bo_ref, iou_ref, tb_ref):
    b = pl.program_id(0)

    @pl.when(b == 0)
    def _():
        out_ref[0, 0] = 0.0

    cxb = jnp.broadcast_to(cxy_ref[0:1, :], (NGT, NPOS))
    cyb = jnp.broadcast_to(cxy_ref[1:2, :], (NGT, NPOS))

    g = gt_ref[0]
    g0 = g[:, 0:1]
    g1 = g[:, 1:2]
    g2 = g[:, 2:3]
    g3 = g[:, 3:4]
    gx1 = g0 - g2 * 0.5
    gy1 = g1 - g3 * 0.5
    gx2 = g0 + g2 * 0.5
    gy2 = g1 + g3 * 0.5
    area_g = (gx2 - gx1) * (gy2 - gy1)

    j_iota = jax.lax.broadcasted_iota(jnp.int32, (NGT, NPOS), 0).astype(jnp.float32)
    pos_iota = jax.lax.broadcasted_iota(jnp.int32, (NGT, NPOS), 1).astype(jnp.float32)

    dx2 = gx2 - cxb
    dx1 = cxb - gx1
    dy2 = gy2 - cyb
    dy1 = cyb - gy1
    gbest = jnp.full((NGT, 1), -1.0, jnp.float32)
    gflat = jnp.zeros((NGT, 1), jnp.float32)
    for a in range(NA):
        awh = anch_ref[a, 0] * 0.5
        ahh = anch_ref[a, 1] * 0.5
        area_col = area_g + (anch_ref[a, 0] * anch_ref[a, 1] + 1e-10)
        iw = jnp.maximum(jnp.minimum(dx2, awh) + jnp.minimum(dx1, awh), 0.0)
        ih = jnp.maximum(jnp.minimum(dy2, ahh) + jnp.minimum(dy1, ahh), 0.0)
        inter = iw * ih
        iou = inter * pl.reciprocal(area_col - inter, approx=True)
        iou_ref[a, 0:NGT, :] = iou

        m = jnp.max(iou, axis=1, keepdims=True)
        pidx = jnp.min(jnp.where(iou == m, pos_iota, BIG), axis=1,
                       keepdims=True)
        flat = pidx + float(a * NPOS)
        upd = m > gbest
        gbest = jnp.where(upd, m, gbest)
        gflat = jnp.where(upd, flat, gflat)

    gT = gtt_ref[0]
    g_box = gT[0:4, :].astype(jnp.bfloat16)
    clscol = g[:, 4:5]
    c_row = jax.lax.broadcasted_iota(jnp.int32, (NGT, NC), 1).astype(jnp.float32)
    perm = jnp.where(clscol == c_row, 1.0, 0.0).astype(jnp.bfloat16)
    ones_row = jnp.ones((1, NGT), jnp.bfloat16)
    ones_perm = jnp.concatenate([ones_row, perm], axis=0)
    scat_val = 3.0 + 0.01 * j_iota

    acc1 = jnp.zeros((1, NPOS), jnp.float32)

    for a in range(NA):
        gflat_a = gflat - float(a * NPOS)
        eqm = gflat_a == pos_iota
        iou_s = jnp.where(eqm, scat_val, iou_ref[a, 0:NGT, :])
        bo = jnp.max(iou_s, axis=0, keepdims=True)
        bo_ref[a:a + 1, :] = bo
        matchf = (bo > IOU_TH).astype(jnp.float32)

        onehot = jnp.where(iou_s == bo, 1.0, 0.0).astype(jnp.bfloat16)
        t_box = _mm(g_box, onehot)
        tb_ref[a, 0:4, :] = t_box

        e_bf = jnp.exp(pred_ref[0, a * CH + 5:a * CH + CH, :])
        sg = _mm(ones_perm, e_bf)
        s_raw = sg[0:1, :]
        e_gt = sg[1:21, :]
        s2_raw = _mm(ones_row, e_bf * e_bf)
        te_raw = _mm(ones_row, e_gt.astype(jnp.bfloat16) * onehot)
        s = jnp.where(matchf > 0.0, s_raw, float(NC))
        s2 = jnp.where(matchf > 0.0, s2_raw, float(NC))
        esel = jnp.where(matchf > 0.0, te_raw, 1.0)
        r = pl.reciprocal(s, approx=True)
        acc1 = acc1 + ((s2 * r - 2.0 * esel) * r + 1.0)

    awcol = anchv_ref[0][:, 0:1]
    ahcol = anchv_ref[0][:, 1:2]

    def _rows(k):
        return jnp.concatenate(
            [pred_ref[0, a * CH + k:a * CH + k + 1, :] for a in range(NA)],
            axis=0).astype(jnp.float32)

    p0 = jax.nn.sigmoid(_rows(0))
    p1 = jax.nn.sigmoid(_rows(1))
    p2 = jnp.exp(_rows(2)) * awcol
    p3 = jnp.exp(_rows(3)) * ahcol
    p4 = jax.nn.sigmoid(_rows(4))
    t0 = tb_ref[:, 0, :]
    t1 = tb_ref[:, 1, :]
    t2 = tb_ref[:, 2, :]
    t3 = tb_ref[:, 3, :]
    matchf = (bo_ref[0:NA, :] > IOU_TH).astype(jnp.float32)
    negf = 1.0 - matchf
    t0m = t0 * matchf
    t1m = t1 * matchf
    t2m = t2 * matchf
    t3m = t3 * matchf

    iw = jnp.maximum(
        jnp.minimum(p0 + p2 * 0.5, t0m + t2m * 0.5)
        - jnp.maximum(p0 - p2 * 0.5, t0m - t2m * 0.5), 0.0)
    ih = jnp.maximum(
        jnp.minimum(p1 + p3 * 0.5, t1m + t3m * 0.5)
        - jnp.maximum(p1 - p3 * 0.5, t1m - t3m * 0.5), 0.0)
    inter = iw * ih
    iou_pt = inter * pl.reciprocal(p2 * p3 + t2m * t3m - inter + EPS,
                                   approx=True)

    noobj_c = jnp.where(iou_pt <= IOU_TH, p4 * p4, 0.0)
    prior_c = negf * ((p0 - 0.5 / GX) ** 2 + (p1 - 0.5 / GY) ** 2
                      + (p2 - awcol) ** 2 + (p3 - ahcol) ** 2)
    box_c = matchf * ((p0 - t0) ** 2 + (p1 - t1) ** 2
                      + (p2 - t2) ** 2 + (p3 - t3) ** 2)
    obj_c = matchf * (p4 - iou_pt) ** 2

    seen_lt = (seen_ref[0, 0] < 12800).astype(jnp.float32)
    acc5 = noobj_c + box_c + L_OBJ * obj_c + (L_PRIOR * seen_lt) * prior_c

    out_ref[0, 0] += jnp.sum(acc5) + jnp.sum(acc1)


def _run(pred_r, gt, gt_t, anchv, cxy, anch, seen_arr, interpret=False):
    B = pred_r.shape[0]
    return pl.pallas_call(
        _body,
        grid=(B,),
        in_specs=[
            pl.BlockSpec((1, NA * CH, NPOS), lambda b: (b, 0, 0)),
            pl.BlockSpec((1, NGT, 5), lambda b: (b, 0, 0)),
            pl.BlockSpec((1, 5, NGT), lambda b: (b, 0, 0)),
            pl.BlockSpec((1, NA, 2), lambda b: (0, 0, 0)),
            pl.BlockSpec((2, NPOS), lambda b: (0, 0)),
            pl.BlockSpec(memory_space=pltpu.SMEM),
            pl.BlockSpec(memory_space=pltpu.SMEM),
        ],
        out_specs=pl.BlockSpec(memory_space=pltpu.SMEM),
        out_shape=jax.ShapeDtypeStruct((1, 1), jnp.float32),
        scratch_shapes=[
            pltpu.VMEM((8, NPOS), jnp.float32),
            pltpu.VMEM((NA, NGT, NPOS), jnp.float32),
            pltpu.VMEM((NA, 8, NPOS), jnp.float32),
        ],
        interpret=interpret,
    )(pred_r, gt, gt_t, anchv, cxy, anch, seen_arr)


def kernel(prediction, groundtruth, anchors, seen, interpret=False):
    B = prediction.shape[0]
    pred_r = prediction.astype(jnp.bfloat16).reshape(B, NA * CH, NPOS)
    gt = groundtruth
    gt_t = jnp.transpose(groundtruth, (0, 2, 1))
    xs = (jnp.arange(GX, dtype=jnp.float32) + 0.5) / GX
    ys = (jnp.arange(GY, dtype=jnp.float32) + 0.5) / GY
    cx = jnp.tile(xs, (GY,))
    cy = jnp.repeat(ys, GX)
    cxy = jnp.stack([cx, cy], axis=0)
    anch = anchors.reshape(NA, 2)
    anchv = anch.reshape(1, NA, 2)
    seen_arr = jnp.asarray(seen, jnp.int32).reshape(1, 1)
    out = _run(pred_r, gt, gt_t, anchv, cxy, anch, seen_arr,
               interpret=interpret)
    return out[0, 0]


# speedup vs baseline: 1.0671x; 1.0671x over previous
import jax
import jax.numpy as jnp
from jax.experimental import pallas as pl
from jax.experimental.pallas import tpu as pltpu

NA = 5
NC = 20
NGT = 20
GY = 52
GX = 52
NPOS = GY * GX
CH = 25
IOU_TH = 0.6
L_OBJ = 5.0
L_PRIOR = 0.01
EPS = 1e-05
BIG = 1e9


def _mm(lhs_bf, rhs_bf):
    return jax.lax.dot_general(lhs_bf, rhs_bf, (((1,), (0,)), ((), ())),
                               preferred_element_type=jnp.float32)


def _body(pred_ref, gt_ref, gtt_ref, anchv_ref, cxy_ref, anch_ref, seen_ref,
          out_ref, bo_ref, iou_ref, tb_ref):
    b = pl.program_id(0)

    @pl.when(b == 0)
    def _():
        out_ref[0, 0] = 0.0

    cxb = jnp.broadcast_to(cxy_ref[0:1, :], (NGT, NPOS))
    cyb = jnp.broadcast_to(cxy_ref[1:2, :], (NGT, NPOS))

    g = gt_ref[0]
    g0 = g[:, 0:1]
    g1 = g[:, 1:2]
    g2 = g[:, 2:3]
    g3 = g[:, 3:4]
    gx1 = g0 - g2 * 0.5
    gy1 = g1 - g3 * 0.5
    gx2 = g0 + g2 * 0.5
    gy2 = g1 + g3 * 0.5
    area_g = (gx2 - gx1) * (gy2 - gy1)

    j_iota = jax.lax.broadcasted_iota(jnp.int32, (NGT, NPOS), 0).astype(jnp.float32)
    pos_iota = jax.lax.broadcasted_iota(jnp.int32, (NGT, NPOS), 1).astype(jnp.float32)

    dx2 = gx2 - cxb
    dx1 = cxb - gx1
    dy2 = gy2 - cyb
    dy1 = cyb - gy1
    gbest = jnp.full((NGT, 1), -1.0, jnp.float32)
    gflat = jnp.zeros((NGT, 1), jnp.float32)
    for a in range(NA):
        awh = anch_ref[a, 0] * 0.5
        ahh = anch_ref[a, 1] * 0.5
        area_col = area_g + (anch_ref[a, 0] * anch_ref[a, 1] + 1e-10)
        iw = jnp.maximum(jnp.minimum(dx2, awh) + jnp.minimum(dx1, awh), 0.0)
        ih = jnp.maximum(jnp.minimum(dy2, ahh) + jnp.minimum(dy1, ahh), 0.0)
        inter = iw * ih
        iou = inter * pl.reciprocal(area_col - inter, approx=True)
        iou_ref[a, 0:NGT, :] = iou

        m = jnp.max(iou, axis=1, keepdims=True)
        pidx = jnp.min(jnp.where(iou == m, pos_iota, BIG), axis=1,
                       keepdims=True)
        flat = pidx + float(a * NPOS)
        upd = m > gbest
        gbest = jnp.where(upd, m, gbest)
        gflat = jnp.where(upd, flat, gflat)

    gT = gtt_ref[0]
    g_box = gT[0:4, :].astype(jnp.bfloat16)
    clscol = g[:, 4:5]
    c_row = jax.lax.broadcasted_iota(jnp.int32, (NGT, NC), 1).astype(jnp.float32)
    perm = jnp.where(clscol == c_row, 1.0, 0.0).astype(jnp.bfloat16)
    ones_row = jnp.ones((1, NGT), jnp.bfloat16)
    ones_perm = jnp.concatenate([ones_row, perm], axis=0)
    scat_val = 3.0 + 0.01 * j_iota

    acc1 = jnp.zeros((1, NPOS), jnp.float32)

    for a in range(NA):
        gflat_a = gflat - float(a * NPOS)
        eqm = gflat_a == pos_iota
        iou_s = jnp.where(eqm, scat_val, iou_ref[a, 0:NGT, :])
        bo = jnp.max(iou_s, axis=0, keepdims=True)
        bo_ref[a:a + 1, :] = bo
        matchf = (bo > IOU_TH).astype(jnp.float32)

        onehot = jnp.where(iou_s == bo, 1.0, 0.0).astype(jnp.bfloat16)
        t_box = _mm(g_box, onehot)
        tb_ref[a, 0:4, :] = t_box

        e_bf = jnp.exp(pred_ref[0, a * CH + 5:a * CH + CH, :]).astype(jnp.bfloat16)
        sg = _mm(ones_perm, e_bf)
        s_raw = sg[0:1, :]
        e_gt = sg[1:21, :]
        s2_raw = _mm(ones_row, e_bf * e_bf)
        te_raw = _mm(ones_row, e_gt.astype(jnp.bfloat16) * onehot)
        s = jnp.where(matchf > 0.0, s_raw, float(NC))
        s2 = jnp.where(matchf > 0.0, s2_raw, float(NC))
        esel = jnp.where(matchf > 0.0, te_raw, 1.0)
        r = pl.reciprocal(s, approx=True)
        acc1 = acc1 + ((s2 * r - 2.0 * esel) * r + 1.0)

    awcol = anchv_ref[0][:, 0:1]
    ahcol = anchv_ref[0][:, 1:2]

    def _rows(k):
        return jnp.concatenate(
            [pred_ref[0, a * CH + k:a * CH + k + 1, :] for a in range(NA)],
            axis=0).astype(jnp.float32)

    p0 = jax.nn.sigmoid(_rows(0))
    p1 = jax.nn.sigmoid(_rows(1))
    p2 = jnp.exp(_rows(2)) * awcol
    p3 = jnp.exp(_rows(3)) * ahcol
    p4 = jax.nn.sigmoid(_rows(4))
    t0 = tb_ref[:, 0, :]
    t1 = tb_ref[:, 1, :]
    t2 = tb_ref[:, 2, :]
    t3 = tb_ref[:, 3, :]
    matchf = (bo_ref[0:NA, :] > IOU_TH).astype(jnp.float32)
    negf = 1.0 - matchf
    t0m = t0 * matchf
    t1m = t1 * matchf
    t2m = t2 * matchf
    t3m = t3 * matchf

    iw = jnp.maximum(
        jnp.minimum(p0 + p2 * 0.5, t0m + t2m * 0.5)
        - jnp.maximum(p0 - p2 * 0.5, t0m - t2m * 0.5), 0.0)
    ih = jnp.maximum(
        jnp.minimum(p1 + p3 * 0.5, t1m + t3m * 0.5)
        - jnp.maximum(p1 - p3 * 0.5, t1m - t3m * 0.5), 0.0)
    inter = iw * ih
    iou_pt = inter * pl.reciprocal(p2 * p3 + t2m * t3m - inter + EPS,
                                   approx=True)

    noobj_c = jnp.where(iou_pt <= IOU_TH, p4 * p4, 0.0)
    prior_c = negf * ((p0 - 0.5 / GX) ** 2 + (p1 - 0.5 / GY) ** 2
                      + (p2 - awcol) ** 2 + (p3 - ahcol) ** 2)
    box_c = matchf * ((p0 - t0) ** 2 + (p1 - t1) ** 2
                      + (p2 - t2) ** 2 + (p3 - t3) ** 2)
    obj_c = matchf * (p4 - iou_pt) ** 2

    seen_lt = (seen_ref[0, 0] < 12800).astype(jnp.float32)
    acc5 = noobj_c + box_c + L_OBJ * obj_c + (L_PRIOR * seen_lt) * prior_c

    out_ref[0, 0] += jnp.sum(acc5) + jnp.sum(acc1)


def _run(pred_r, gt, gt_t, anchv, cxy, anch, seen_arr, interpret=False):
    B = pred_r.shape[0]
    return pl.pallas_call(
        _body,
        grid=(B,),
        in_specs=[
            pl.BlockSpec((1, NA * CH, NPOS), lambda b: (b, 0, 0)),
            pl.BlockSpec((1, NGT, 5), lambda b: (b, 0, 0)),
            pl.BlockSpec((1, 5, NGT), lambda b: (b, 0, 0)),
            pl.BlockSpec((1, NA, 2), lambda b: (0, 0, 0)),
            pl.BlockSpec((2, NPOS), lambda b: (0, 0)),
            pl.BlockSpec(memory_space=pltpu.SMEM),
            pl.BlockSpec(memory_space=pltpu.SMEM),
        ],
        out_specs=pl.BlockSpec(memory_space=pltpu.SMEM),
        out_shape=jax.ShapeDtypeStruct((1, 1), jnp.float32),
        scratch_shapes=[
            pltpu.VMEM((8, NPOS), jnp.float32),
            pltpu.VMEM((NA, NGT, NPOS), jnp.float32),
            pltpu.VMEM((NA, 8, NPOS), jnp.float32),
        ],
        interpret=interpret,
    )(pred_r, gt, gt_t, anchv, cxy, anch, seen_arr)


def kernel(prediction, groundtruth, anchors, seen, interpret=False):
    B = prediction.shape[0]
    pred_r = prediction.reshape(B, NA * CH, NPOS)
    gt = groundtruth
    gt_t = jnp.transpose(groundtruth, (0, 2, 1))
    xs = (jnp.arange(GX, dtype=jnp.float32) + 0.5) / GX
    ys = (jnp.arange(GY, dtype=jnp.float32) + 0.5) / GY
    cx = jnp.tile(xs, (GY,))
    cy = jnp.repeat(ys, GX)
    cxy = jnp.stack([cx, cy], axis=0)
    anch = anchors.reshape(NA, 2)
    anchv = anch.reshape(1, NA, 2)
    seen_arr = jnp.asarray(seen, jnp.int32).reshape(1, 1)
    out = _run(pred_r, gt, gt_t, anchv, cxy, anch, seen_arr,
               interpret=interpret)
    return out[0, 0]


# speedup vs baseline: 1.1081x; 1.0385x over previous
import jax
import jax.numpy as jnp
from jax.experimental import pallas as pl
from jax.experimental.pallas import tpu as pltpu

NA = 5
NC = 20
NGT = 20
GY = 52
GX = 52
NPOS = GY * GX
CH = 25
IOU_TH = 0.6
L_OBJ = 5.0
L_PRIOR = 0.01
EPS = 1e-05
BIG = 1e9


def _mm(lhs_bf, rhs_bf):
    return jax.lax.dot_general(lhs_bf, rhs_bf, (((1,), (0,)), ((), ())),
                               preferred_element_type=jnp.float32)


def _body(pred_ref, gt_ref, gtt_ref, anchv_ref, cxy_ref, anch_ref, seen_ref,
          out_ref, bo_ref, iou_ref, tb_ref):
    b = pl.program_id(0)

    @pl.when(b == 0)
    def _():
        out_ref[0, 0] = 0.0

    cxb = jnp.broadcast_to(cxy_ref[0:1, :], (NGT, NPOS))
    cyb = jnp.broadcast_to(cxy_ref[1:2, :], (NGT, NPOS))

    g = gt_ref[0]
    g0 = g[:, 0:1]
    g1 = g[:, 1:2]
    g2 = g[:, 2:3]
    g3 = g[:, 3:4]
    gx1 = g0 - g2 * 0.5
    gy1 = g1 - g3 * 0.5
    gx2 = g0 + g2 * 0.5
    gy2 = g1 + g3 * 0.5
    area_g = (gx2 - gx1) * (gy2 - gy1)

    j_iota = jax.lax.broadcasted_iota(jnp.int32, (NGT, NPOS), 0).astype(jnp.float32)
    pos_iota = jax.lax.broadcasted_iota(jnp.int32, (NGT, NPOS), 1).astype(jnp.float32)

    dx2 = gx2 - cxb
    dx1 = cxb - gx1
    dy2 = gy2 - cyb
    dy1 = cyb - gy1
    gbest = jnp.full((NGT, 1), -1.0, jnp.float32)
    gflat = jnp.zeros((NGT, 1), jnp.float32)
    for a in range(NA):
        awh = anch_ref[a, 0] * 0.5
        ahh = anch_ref[a, 1] * 0.5
        area_col = area_g + (anch_ref[a, 0] * anch_ref[a, 1] + 1e-10)
        iw = jnp.maximum(jnp.minimum(dx2, awh) + jnp.minimum(dx1, awh), 0.0)
        ih = jnp.maximum(jnp.minimum(dy2, ahh) + jnp.minimum(dy1, ahh), 0.0)
        inter = iw * ih
        iou = inter * pl.reciprocal(area_col - inter, approx=True)
        iou_ref[a, 0:NGT, :] = iou

        m = jnp.max(iou, axis=1, keepdims=True)
        pidx = jnp.min(jnp.where(iou == m, pos_iota, BIG), axis=1,
                       keepdims=True)
        flat = pidx + float(a * NPOS)
        upd = m > gbest
        gbest = jnp.where(upd, m, gbest)
        gflat = jnp.where(upd, flat, gflat)

    gT = gtt_ref[0]
    g_box = gT[0:4, :].astype(jnp.bfloat16)
    clscol = g[:, 4:5]
    c_row = jax.lax.broadcasted_iota(jnp.int32, (NGT, NC), 1).astype(jnp.float32)
    perm = jnp.where(clscol == c_row, 1.0, 0.0).astype(jnp.bfloat16)
    ones_row = jnp.ones((1, NGT), jnp.bfloat16)
    perm_ones = jnp.concatenate([perm, ones_row], axis=0)
    scat_val = 3.0 + 0.01 * j_iota

    acc1 = jnp.zeros((1, NPOS), jnp.float32)

    for a in range(NA):
        gflat_a = gflat - float(a * NPOS)
        eqm = gflat_a == pos_iota
        iou_s = jnp.where(eqm, scat_val, iou_ref[a, 0:NGT, :])
        bo = jnp.max(iou_s, axis=0, keepdims=True)
        bo_ref[a:a + 1, :] = bo
        matchf = (bo > IOU_TH).astype(jnp.float32)

        onehot = jnp.where(iou_s == bo, 1.0, 0.0).astype(jnp.bfloat16)
        t_box = _mm(g_box, onehot)
        tb_ref[a, 0:4, :] = t_box

        e_bf = jnp.exp(pred_ref[0, a * CH + 5:a * CH + CH, :]).astype(jnp.bfloat16)
        sg = _mm(perm_ones, e_bf)
        e_gt = sg[0:NGT, :]
        s_raw = sg[NGT:NGT + 1, :]
        s2_raw = _mm(ones_row, e_bf * e_bf)
        te_raw = _mm(ones_row, e_gt.astype(jnp.bfloat16) * onehot)
        s = jnp.where(matchf > 0.0, s_raw, float(NC))
        s2 = jnp.where(matchf > 0.0, s2_raw, float(NC))
        esel = jnp.where(matchf > 0.0, te_raw, 1.0)
        r = pl.reciprocal(s, approx=True)
        acc1 = acc1 + ((s2 * r - 2.0 * esel) * r + 1.0)

    awcol = anchv_ref[0][:, 0:1]
    ahcol = anchv_ref[0][:, 1:2]

    def _rows(k):
        return jnp.concatenate(
            [pred_ref[0, a * CH + k:a * CH + k + 1, :] for a in range(NA)],
            axis=0).astype(jnp.float32)

    p0 = jax.nn.sigmoid(_rows(0))
    p1 = jax.nn.sigmoid(_rows(1))
    p2 = jnp.exp(_rows(2)) * awcol
    p3 = jnp.exp(_rows(3)) * ahcol
    p4 = jax.nn.sigmoid(_rows(4))
    t0 = tb_ref[:, 0, :]
    t1 = tb_ref[:, 1, :]
    t2 = tb_ref[:, 2, :]
    t3 = tb_ref[:, 3, :]
    matchf = (bo_ref[0:NA, :] > IOU_TH).astype(jnp.float32)
    negf = 1.0 - matchf
    t0m = t0 * matchf
    t1m = t1 * matchf
    t2m = t2 * matchf
    t3m = t3 * matchf

    iw = jnp.maximum(
        jnp.minimum(p0 + p2 * 0.5, t0m + t2m * 0.5)
        - jnp.maximum(p0 - p2 * 0.5, t0m - t2m * 0.5), 0.0)
    ih = jnp.maximum(
        jnp.minimum(p1 + p3 * 0.5, t1m + t3m * 0.5)
        - jnp.maximum(p1 - p3 * 0.5, t1m - t3m * 0.5), 0.0)
    inter = iw * ih
    iou_pt = inter * pl.reciprocal(p2 * p3 + t2m * t3m - inter + EPS,
                                   approx=True)

    noobj_c = jnp.where(iou_pt <= IOU_TH, p4 * p4, 0.0)
    prior_c = negf * ((p0 - 0.5 / GX) ** 2 + (p1 - 0.5 / GY) ** 2
                      + (p2 - awcol) ** 2 + (p3 - ahcol) ** 2)
    box_c = matchf * ((p0 - t0) ** 2 + (p1 - t1) ** 2
                      + (p2 - t2) ** 2 + (p3 - t3) ** 2)
    obj_c = matchf * (p4 - iou_pt) ** 2

    seen_lt = (seen_ref[0, 0] < 12800).astype(jnp.float32)
    acc5 = noobj_c + box_c + L_OBJ * obj_c + (L_PRIOR * seen_lt) * prior_c

    out_ref[0, 0] += jnp.sum(acc5) + jnp.sum(acc1)


def _run(pred_r, gt, gt_t, anchv, cxy, anch, seen_arr, interpret=False):
    B = pred_r.shape[0]
    return pl.pallas_call(
        _body,
        grid=(B,),
        in_specs=[
            pl.BlockSpec((1, NA * CH, NPOS), lambda b: (b, 0, 0)),
            pl.BlockSpec((1, NGT, 5), lambda b: (b, 0, 0)),
            pl.BlockSpec((1, 5, NGT), lambda b: (b, 0, 0)),
            pl.BlockSpec((1, NA, 2), lambda b: (0, 0, 0)),
            pl.BlockSpec((2, NPOS), lambda b: (0, 0)),
            pl.BlockSpec(memory_space=pltpu.SMEM),
            pl.BlockSpec(memory_space=pltpu.SMEM),
        ],
        out_specs=pl.BlockSpec(memory_space=pltpu.SMEM),
        out_shape=jax.ShapeDtypeStruct((1, 1), jnp.float32),
        scratch_shapes=[
            pltpu.VMEM((8, NPOS), jnp.float32),
            pltpu.VMEM((NA, NGT, NPOS), jnp.float32),
            pltpu.VMEM((NA, 8, NPOS), jnp.float32),
        ],
        interpret=interpret,
    )(pred_r, gt, gt_t, anchv, cxy, anch, seen_arr)


def kernel(prediction, groundtruth, anchors, seen, interpret=False):
    B = prediction.shape[0]
    pred_r = prediction.reshape(B, NA * CH, NPOS)
    gt = groundtruth
    gt_t = jnp.transpose(groundtruth, (0, 2, 1))
    xs = (jnp.arange(GX, dtype=jnp.float32) + 0.5) / GX
    ys = (jnp.arange(GY, dtype=jnp.float32) + 0.5) / GY
    cx = jnp.tile(xs, (GY,))
    cy = jnp.repeat(ys, GX)
    cxy = jnp.stack([cx, cy], axis=0)
    anch = anchors.reshape(NA, 2)
    anchv = anch.reshape(1, NA, 2)
    seen_arr = jnp.asarray(seen, jnp.int32).reshape(1, 1)
    out = _run(pred_r, gt, gt_t, anchv, cxy, anch, seen_arr,
               interpret=interpret)
    return out[0, 0]
